# baseline (speedup 1.0000x reference)
import jax
import jax.numpy as jnp
from jax import lax
from jax.experimental import pallas as pl
from jax.experimental.pallas import tpu as pltpu
from jax.experimental.pallas import tpu_sc as plsc

N = 100000
D = 32
DH = 16
E = 1600000
CHUNK = 128
C2 = 256
SUBS = 8
NTILES = 16
NSUP = 102
STEPS = 24
KITER = (4 * NSUP) // STEPS
NSUP_TOT = NTILES * NSUP
EPAD = NSUP_TOT * SUBS * CHUNK
ACC_ROWS = 100096
RPT = ACC_ROWS // NTILES
ZROWS = 782

_mesh = plsc.VectorSubcoreMesh(core_axis_name="c", subcore_axis_name="s")
_sc_params = pltpu.CompilerParams(use_tc_tiling_on_sc=False)


def _zero_acc(zbuf, acc, t):
    def fill_zero(i, carry):
        zbuf[i, :] = jnp.zeros((DH,), jnp.float32)
        return carry

    lax.fori_loop(0, ZROWS, fill_zero, 0)

    def zero_copy(k, carry):
        pltpu.sync_copy(zbuf, acc.at[pl.ds(t * RPT + k * ZROWS, ZROWS)])
        return carry

    lax.fori_loop(0, RPT // ZROWS, zero_copy, 0)


def _deg_body(eidx_r_hbm, eidx_b_hbm, deg_r_hbm, deg_b_hbm,
              eA, eB, ones_v, zbuf, acc, isemA, isemB, ssemA, ssemB):
    c = lax.axis_index("c")
    t = lax.axis_index("s")

    def fill_ones(i, carry):
        ones_v[i, :] = jnp.ones((DH,), jnp.float32)
        return carry

    lax.fori_loop(0, C2, fill_ones, 0)
    _zero_acc(zbuf, acc, t)
    plsc.subcore_barrier()

    DKIT = NSUP // 2

    def run(eidx_hbm):
        base = t * NSUP

        def scatters(e, sem):
            for m in range(4):
                pltpu.async_copy(ones_v, acc.at[e.at[4 + m]], sem, add=True)

        def wait_scatters(e, sem):
            for m in range(4):
                pltpu.make_async_copy(ones_v, acc.at[e.at[4 + m]],
                                      sem).wait()

        pltpu.sync_copy(eidx_hbm.at[base], eA)

        def body(k, carry):
            gB = 2 * k + 1

            @pl.when(k > 0)
            def _():
                wait_scatters(eB, ssemB)

            idx_b = pltpu.async_copy(eidx_hbm.at[base + gB], eB, isemB)
            scatters(eA, ssemA)
            idx_b.wait()
            wait_scatters(eA, ssemA)

            @pl.when(k < DKIT - 1)
            def _():
                pltpu.async_copy(eidx_hbm.at[base + gB + 1], eA, isemA)

            scatters(eB, ssemB)

            @pl.when(k < DKIT - 1)
            def _():
                pltpu.make_async_copy(eidx_hbm.at[base + gB + 1], eA,
                                      isemA).wait()

            return carry

        lax.fori_loop(0, DKIT, body, 0)
        wait_scatters(eB, ssemB)

    @pl.when(c == 0)
    def _():
        run(eidx_r_hbm)

    @pl.when(c == 1)
    def _():
        run(eidx_b_hbm)

    plsc.subcore_barrier()

    @pl.when(c == 0)
    def _():
        pltpu.sync_copy(acc.at[pl.ds(t * RPT, RPT)],
                        deg_r_hbm.at[pl.ds(t * RPT, RPT)])

    @pl.when(c == 1)
    def _():
        pltpu.sync_copy(acc.at[pl.ds(t * RPT, RPT)],
                        deg_b_hbm.at[pl.ds(t * RPT, RPT)])


_deg_call = pl.kernel(
    _deg_body,
    out_type=(jax.ShapeDtypeStruct((ACC_ROWS, DH), jnp.float32),
              jax.ShapeDtypeStruct((ACC_ROWS, DH), jnp.float32)),
    mesh=_mesh,
    scratch_types=[
        pltpu.VMEM((SUBS, C2), jnp.int32),
        pltpu.VMEM((SUBS, C2), jnp.int32),
        pltpu.VMEM((C2, DH), jnp.float32),
        pltpu.VMEM((ZROWS, DH), jnp.float32),
        pltpu.VMEM_SHARED((ACC_ROWS, DH), jnp.float32),
        pltpu.SemaphoreType.DMA,
        pltpu.SemaphoreType.DMA,
        pltpu.SemaphoreType.DMA,
        pltpu.SemaphoreType.DMA,
    ],
    compiler_params=_sc_params,
)


def _agg_body(tlo_hbm, thi_hbm, eidx_hbm, out_lo_hbm, out_hi_hbm,
              e0, e1, e2, rows0, rows1, rows2, zbuf, acc,
              is0, is1, is2, gs0, gs1, gs2, ss0, ss1, ss2):
    c = lax.axis_index("c")
    t = lax.axis_index("s")
    ebuf = (e0, e1, e2)
    isem = (is0, is1, is2)
    rows = (rows0, rows1, rows2)
    gsem = (gs0, gs1, gs2)
    ssem = (ss0, ss1, ss2)
    _zero_acc(zbuf, acc, t)
    plsc.subcore_barrier()

    def run(table):
        base = t * NSUP

        def src_slice(j):
            return ebuf[(j // 4) % 3].at[j % 4]

        def dst_slice(j):
            return ebuf[(j // 4) % 3].at[4 + (j % 4)]

        def g_issue(j):
            p = (j % 24) % 3
            pltpu.async_copy(table.at[src_slice(j % 24)], rows[p], gsem[p])

        def g_wait(j):
            p = (j % 24) % 3
            pltpu.make_async_copy(table.at[src_slice(j % 24)], rows[p],
                                  gsem[p]).wait()

        def s_issue(j):
            p = (j % 24) % 3
            pltpu.async_copy(rows[p], acc.at[dst_slice(j % 24)], ssem[p],
                             add=True)

        def s_wait(j):
            p = (j % 24) % 3
            pltpu.make_async_copy(rows[p], acc.at[dst_slice(j % 24)],
                                  ssem[p]).wait()

        def refill_issue(k, sup):
            pltpu.async_copy(eidx_hbm.at[base + 6 * k + sup], ebuf[sup % 3],
                             isem[sup % 3])

        def refill_wait(k, sup):
            pltpu.make_async_copy(eidx_hbm.at[base + 6 * k + sup],
                                  ebuf[sup % 3], isem[sup % 3]).wait()

        pltpu.sync_copy(eidx_hbm.at[base], e0)
        pltpu.sync_copy(eidx_hbm.at[base + 1], e1)
        pltpu.sync_copy(eidx_hbm.at[base + 2], e2)
        g_issue(0)

        def body(k, carry):
            for j in range(STEPS):
                if j < 2:
                    @pl.when(k > 0)
                    def _(j=j):
                        s_wait(j - 2)
                else:
                    s_wait(j - 2)
                if j == 2:
                    @pl.when(k > 0)
                    def _():
                        refill_issue(k, 2)
                elif j in (6, 10, 14):
                    refill_issue(k, (j + 6) // 4)
                elif j in (18, 22):
                    @pl.when(k < KITER - 1)
                    def _(j=j):
                        refill_issue(k, (j + 6) // 4)
                if j in (3, 7):
                    @pl.when(k > 0)
                    def _(j=j):
                        refill_wait(k, (j + 1) // 4)
                elif j in (11, 15, 19):
                    refill_wait(k, (j + 1) // 4)
                elif j == 23:
                    @pl.when(k < KITER - 1)
                    def _():
                        refill_wait(k, 6)
                if j < STEPS - 1:
                    g_issue(j + 1)
                else:
                    @pl.when(k < KITER - 1)
                    def _():
                        g_issue(0)
                g_wait(j)
                s_issue(j)
            return carry

        lax.fori_loop(0, KITER, body, 0)
        s_wait(22)
        s_wait(23)

    @pl.when(c == 0)
    def _():
        run(tlo_hbm)

    @pl.when(c == 1)
    def _():
        run(thi_hbm)

    plsc.subcore_barrier()

    @pl.when(c == 0)
    def _():
        pltpu.sync_copy(acc.at[pl.ds(t * RPT, RPT)],
                        out_lo_hbm.at[pl.ds(t * RPT, RPT)])

    @pl.when(c == 1)
    def _():
        pltpu.sync_copy(acc.at[pl.ds(t * RPT, RPT)],
                        out_hi_hbm.at[pl.ds(t * RPT, RPT)])


_agg_call = pl.kernel(
    _agg_body,
    out_type=(jax.ShapeDtypeStruct((ACC_ROWS, DH), jnp.float32),
              jax.ShapeDtypeStruct((ACC_ROWS, DH), jnp.float32)),
    mesh=_mesh,
    scratch_types=[
        pltpu.VMEM((SUBS, C2), jnp.int32),
        pltpu.VMEM((SUBS, C2), jnp.int32),
        pltpu.VMEM((SUBS, C2), jnp.int32),
        pltpu.VMEM((C2, DH), jnp.float32),
        pltpu.VMEM((C2, DH), jnp.float32),
        pltpu.VMEM((C2, DH), jnp.float32),
        pltpu.VMEM((ZROWS, DH), jnp.float32),
        pltpu.VMEM_SHARED((ACC_ROWS, DH), jnp.float32),
    ] + [pltpu.SemaphoreType.DMA] * 9,
    compiler_params=_sc_params,
)

ROWS_TC = 2000
GRID_TC = N // ROWS_TC


def _tc1_body(x_ref, lo_ref, hi_ref, deg_ref, ws_ref, wl_ref, wh_ref, b_ref,
              out_lo_ref, out_hi_ref):
    agg = (jnp.dot(lo_ref[...], wl_ref[...], preferred_element_type=jnp.float32)
           + jnp.dot(hi_ref[...], wh_ref[...], preferred_element_type=jnp.float32))
    inv = 1.0 / jnp.maximum(deg_ref[...][:, :1], 1.0)
    h = (jnp.dot(x_ref[...], ws_ref[...], preferred_element_type=jnp.float32)
         + inv * agg + b_ref[...])
    h = jnp.maximum(h, 0.0)
    out_lo_ref[...] = h[:, :DH]
    out_hi_ref[...] = h[:, DH:]


def _tc2_body(xlo_ref, xhi_ref, lo_ref, hi_ref, deg_ref,
              wslo_ref, wshi_ref, wl_ref, wh_ref, b_ref, out_ref):
    agg = (jnp.dot(lo_ref[...], wl_ref[...], preferred_element_type=jnp.float32)
           + jnp.dot(hi_ref[...], wh_ref[...], preferred_element_type=jnp.float32))
    inv = 1.0 / jnp.maximum(deg_ref[...][:, :1], 1.0)
    h = (jnp.dot(xlo_ref[...], wslo_ref[...], preferred_element_type=jnp.float32)
         + jnp.dot(xhi_ref[...], wshi_ref[...], preferred_element_type=jnp.float32)
         + inv * agg + b_ref[...])
    out_ref[...] = h


def _row_spec(cols):
    return pl.BlockSpec((ROWS_TC, cols), lambda i: (i, 0))


def _full_spec(r, c):
    return pl.BlockSpec((r, c), lambda i: (0, 0))


_tc1_call = pl.pallas_call(
    _tc1_body,
    grid=(GRID_TC,),
    in_specs=[_row_spec(D), _row_spec(DH), _row_spec(DH), _row_spec(DH),
              _full_spec(D, D), _full_spec(DH, D), _full_spec(DH, D),
              _full_spec(1, D)],
    out_specs=(_row_spec(DH), _row_spec(DH)),
    out_shape=(jax.ShapeDtypeStruct((N, DH), jnp.float32),
               jax.ShapeDtypeStruct((N, DH), jnp.float32)),
)

_tc2_call = pl.pallas_call(
    _tc2_body,
    grid=(GRID_TC,),
    in_specs=[_row_spec(DH), _row_spec(DH), _row_spec(DH), _row_spec(DH),
              _row_spec(DH), _full_spec(DH, D), _full_spec(DH, D),
              _full_spec(DH, D), _full_spec(DH, D), _full_spec(1, D)],
    out_specs=_row_spec(D),
    out_shape=jax.ShapeDtypeStruct((N, D), jnp.float32),
)


def _edge_blocks(ei):
    src = jnp.concatenate([ei[0].astype(jnp.int32),
                           jnp.zeros((EPAD - E,), jnp.int32)])
    dst = jnp.concatenate([ei[1].astype(jnp.int32),
                           jnp.full((EPAD - E,), N, jnp.int32)])
    s3 = src.reshape(NSUP_TOT, 4, C2)
    d3 = dst.reshape(NSUP_TOT, 4, C2)
    return jnp.concatenate([s3, d3], axis=1)


def kernel(x_user, x_item, edge_index_rates, edge_index_rated_by,
           W1_rates_self, W1_rates_neigh, W1_rb_self, W1_rb_neigh,
           W2_rates_self, W2_rates_neigh, W2_rb_self, W2_rb_neigh,
           b1_rates, b1_rb, b2_rates, b2_rb):
    e_r = _edge_blocks(edge_index_rates)
    e_b = _edge_blocks(edge_index_rated_by)

    xu_lo, xu_hi = x_user[:, :DH], x_user[:, DH:]
    xi_lo, xi_hi = x_item[:, :DH], x_item[:, DH:]

    deg_r, deg_b = _deg_call(e_r, e_b)

    b1r = b1_rates.reshape(1, D)
    b1b = b1_rb.reshape(1, D)
    b2r = b2_rates.reshape(1, D)
    b2b = b2_rb.reshape(1, D)

    a1i_lo, a1i_hi = _agg_call(xu_lo, xu_hi, e_r)
    a1u_lo, a1u_hi = _agg_call(xi_lo, xi_hi, e_b)
    hi_lo, hi_hi = _tc1_call(x_item, a1i_lo, a1i_hi, deg_r,
                             W1_rates_self, W1_rates_neigh[:DH],
                             W1_rates_neigh[DH:], b1r)
    hu_lo, hu_hi = _tc1_call(x_user, a1u_lo, a1u_hi, deg_b,
                             W1_rb_self, W1_rb_neigh[:DH],
                             W1_rb_neigh[DH:], b1b)

    a2i_lo, a2i_hi = _agg_call(hu_lo, hu_hi, e_r)
    a2u_lo, a2u_hi = _agg_call(hi_lo, hi_hi, e_b)
    h_item2 = _tc2_call(hi_lo, hi_hi, a2i_lo, a2i_hi, deg_r,
                        W2_rates_self[:DH], W2_rates_self[DH:],
                        W2_rates_neigh[:DH], W2_rates_neigh[DH:], b2r)
    h_user2 = _tc2_call(hu_lo, hu_hi, a2u_lo, a2u_hi, deg_b,
                        W2_rb_self[:DH], W2_rb_self[DH:],
                        W2_rb_neigh[:DH], W2_rb_neigh[DH:], b2b)
    return (h_user2, h_item2)


# speedup vs baseline: 1.0956x; 1.0956x over previous
import jax
import jax.numpy as jnp
from jax import lax
from jax.experimental import pallas as pl
from jax.experimental.pallas import tpu as pltpu
from jax.experimental.pallas import tpu_sc as plsc

N = 100000
D = 32
DH = 16
E = 1600000
CHUNK = 128
SUBS = 8
NTILES = 16
NSUP = 102
STEPS = 48
KITER = (SUBS * NSUP) // STEPS
NSUP_TOT = NTILES * NSUP
EPAD = NSUP_TOT * SUBS * CHUNK
ACC_ROWS = 100096
RPT = ACC_ROWS // NTILES
ZROWS = 782

_mesh = plsc.VectorSubcoreMesh(core_axis_name="c", subcore_axis_name="s")
_sc_params = pltpu.CompilerParams(use_tc_tiling_on_sc=False)


def _zero_acc(zbuf, acc, t):
    def fill_zero(i, carry):
        zbuf[i, :] = jnp.zeros((DH,), jnp.float32)
        return carry

    lax.fori_loop(0, ZROWS, fill_zero, 0)

    def zero_copy(k, carry):
        pltpu.sync_copy(zbuf, acc.at[pl.ds(t * RPT + k * ZROWS, ZROWS)])
        return carry

    lax.fori_loop(0, RPT // ZROWS, zero_copy, 0)


def _deg_body(eidx_r_hbm, eidx_b_hbm, deg_r_hbm, deg_b_hbm,
              eA, eB, ones_v, zbuf, acc, isemA, isemB, ssemA, ssemB):
    c = lax.axis_index("c")
    t = lax.axis_index("s")

    def fill_ones(i, carry):
        ones_v[i, :] = jnp.ones((DH,), jnp.float32)
        return carry

    lax.fori_loop(0, CHUNK, fill_ones, 0)
    _zero_acc(zbuf, acc, t)
    plsc.subcore_barrier()

    DKIT = NSUP // 2

    def run(eidx_hbm):
        base = t * NSUP

        def scatters(e, sem):
            for j in range(SUBS):
                pltpu.async_copy(ones_v, acc.at[e.at[SUBS + j]], sem, add=True)

        def wait_scatters(e, sem):
            for j in range(SUBS):
                pltpu.make_async_copy(ones_v, acc.at[e.at[SUBS + j]],
                                      sem).wait()

        pltpu.sync_copy(eidx_hbm.at[base], eA)

        def body(k, carry):
            gB = 2 * k + 1

            @pl.when(k > 0)
            def _():
                wait_scatters(eB, ssemB)

            idx_b = pltpu.async_copy(eidx_hbm.at[base + gB], eB, isemB)
            scatters(eA, ssemA)
            idx_b.wait()
            wait_scatters(eA, ssemA)

            @pl.when(k < DKIT - 1)
            def _():
                pltpu.async_copy(eidx_hbm.at[base + gB + 1], eA, isemA)

            scatters(eB, ssemB)

            @pl.when(k < DKIT - 1)
            def _():
                pltpu.make_async_copy(eidx_hbm.at[base + gB + 1], eA,
                                      isemA).wait()

            return carry

        lax.fori_loop(0, DKIT, body, 0)
        wait_scatters(eB, ssemB)

    @pl.when(c == 0)
    def _():
        run(eidx_r_hbm)

    @pl.when(c == 1)
    def _():
        run(eidx_b_hbm)

    plsc.subcore_barrier()

    @pl.when(c == 0)
    def _():
        pltpu.sync_copy(acc.at[pl.ds(t * RPT, RPT)],
                        deg_r_hbm.at[pl.ds(t * RPT, RPT)])

    @pl.when(c == 1)
    def _():
        pltpu.sync_copy(acc.at[pl.ds(t * RPT, RPT)],
                        deg_b_hbm.at[pl.ds(t * RPT, RPT)])


_deg_call = pl.kernel(
    _deg_body,
    out_type=(jax.ShapeDtypeStruct((ACC_ROWS, DH), jnp.float32),
              jax.ShapeDtypeStruct((ACC_ROWS, DH), jnp.float32)),
    mesh=_mesh,
    scratch_types=[
        pltpu.VMEM((2 * SUBS, CHUNK), jnp.int32),
        pltpu.VMEM((2 * SUBS, CHUNK), jnp.int32),
        pltpu.VMEM((CHUNK, DH), jnp.float32),
        pltpu.VMEM((ZROWS, DH), jnp.float32),
        pltpu.VMEM_SHARED((ACC_ROWS, DH), jnp.float32),
        pltpu.SemaphoreType.DMA,
        pltpu.SemaphoreType.DMA,
        pltpu.SemaphoreType.DMA,
        pltpu.SemaphoreType.DMA,
    ],
    compiler_params=_sc_params,
)


def _agg_body(tlo_hbm, thi_hbm, eidx_hbm, out_lo_hbm, out_hi_hbm,
              e0, e1, e2, rows0, rows1, rows2, rows3, rows4, rows5, zbuf, acc,
              is0, is1, is2, gs0, gs1, gs2, gs3, gs4, gs5,
              ss0, ss1, ss2, ss3, ss4, ss5):
    c = lax.axis_index("c")
    t = lax.axis_index("s")
    ebuf = (e0, e1, e2)
    isem = (is0, is1, is2)
    rows = (rows0, rows1, rows2, rows3, rows4, rows5)
    gsem = (gs0, gs1, gs2, gs3, gs4, gs5)
    ssem = (ss0, ss1, ss2, ss3, ss4, ss5)
    _zero_acc(zbuf, acc, t)
    plsc.subcore_barrier()

    def run(table):
        base = t * NSUP

        def src_slice(j):
            return ebuf[(j // 8) % 3].at[j % 8]

        def dst_slice(j):
            return ebuf[(j // 8) % 3].at[SUBS + (j % 8)]

        def g_issue(j):
            jj = j % STEPS
            p = jj % 6
            pltpu.async_copy(table.at[src_slice(jj)], rows[p], gsem[p])

        def g_wait(j):
            jj = j % STEPS
            p = jj % 6
            pltpu.make_async_copy(table.at[src_slice(jj)], rows[p],
                                  gsem[p]).wait()

        def s_issue(j):
            jj = j % STEPS
            p = jj % 6
            pltpu.async_copy(rows[p], acc.at[dst_slice(jj)], ssem[p],
                             add=True)

        def s_wait(j):
            jj = j % STEPS
            p = jj % 6
            pltpu.make_async_copy(rows[p], acc.at[dst_slice(jj)],
                                  ssem[p]).wait()

        def refill_issue(k, sup):
            pltpu.async_copy(eidx_hbm.at[base + 6 * k + sup], ebuf[sup % 3],
                             isem[sup % 3])

        def refill_wait(k, sup):
            pltpu.make_async_copy(eidx_hbm.at[base + 6 * k + sup],
                                  ebuf[sup % 3], isem[sup % 3]).wait()

        pltpu.sync_copy(eidx_hbm.at[base], e0)
        pltpu.sync_copy(eidx_hbm.at[base + 1], e1)
        pltpu.sync_copy(eidx_hbm.at[base + 2], e2)
        for j in range(4):
            g_issue(j)

        def body(k, carry):
            for j in range(STEPS):
                if j < 2:
                    @pl.when(k > 0)
                    def _(j=j):
                        s_wait(j - 2)
                else:
                    s_wait(j - 2)
                if j == 2:
                    @pl.when(k > 0)
                    def _():
                        refill_issue(k, 2)
                elif j in (10, 18, 26):
                    refill_issue(k, (j + 14) // 8)
                elif j in (34, 42):
                    @pl.when(k < KITER - 1)
                    def _(j=j):
                        refill_issue(k, (j + 14) // 8)
                if j in (3, 11):
                    @pl.when(k > 0)
                    def _(j=j):
                        refill_wait(k, (j + 5) // 8)
                elif j in (19, 27, 35):
                    refill_wait(k, (j + 5) // 8)
                elif j == 43:
                    @pl.when(k < KITER - 1)
                    def _():
                        refill_wait(k, 6)
                if j < STEPS - 4:
                    g_issue(j + 4)
                else:
                    @pl.when(k < KITER - 1)
                    def _(j=j):
                        g_issue(j + 4)
                g_wait(j)
                s_issue(j)
            return carry

        lax.fori_loop(0, KITER, body, 0)
        s_wait(STEPS - 2)
        s_wait(STEPS - 1)

    @pl.when(c == 0)
    def _():
        run(tlo_hbm)

    @pl.when(c == 1)
    def _():
        run(thi_hbm)

    plsc.subcore_barrier()

    @pl.when(c == 0)
    def _():
        pltpu.sync_copy(acc.at[pl.ds(t * RPT, RPT)],
                        out_lo_hbm.at[pl.ds(t * RPT, RPT)])

    @pl.when(c == 1)
    def _():
        pltpu.sync_copy(acc.at[pl.ds(t * RPT, RPT)],
                        out_hi_hbm.at[pl.ds(t * RPT, RPT)])


_agg_call = pl.kernel(
    _agg_body,
    out_type=(jax.ShapeDtypeStruct((ACC_ROWS, DH), jnp.float32),
              jax.ShapeDtypeStruct((ACC_ROWS, DH), jnp.float32)),
    mesh=_mesh,
    scratch_types=[
        pltpu.VMEM((2 * SUBS, CHUNK), jnp.int32),
        pltpu.VMEM((2 * SUBS, CHUNK), jnp.int32),
        pltpu.VMEM((2 * SUBS, CHUNK), jnp.int32),
        pltpu.VMEM((CHUNK, DH), jnp.float32),
        pltpu.VMEM((CHUNK, DH), jnp.float32),
        pltpu.VMEM((CHUNK, DH), jnp.float32),
        pltpu.VMEM((CHUNK, DH), jnp.float32),
        pltpu.VMEM((CHUNK, DH), jnp.float32),
        pltpu.VMEM((CHUNK, DH), jnp.float32),
        pltpu.VMEM((ZROWS, DH), jnp.float32),
        pltpu.VMEM_SHARED((ACC_ROWS, DH), jnp.float32),
    ] + [pltpu.SemaphoreType.DMA] * 15,
    compiler_params=_sc_params,
)

ROWS_TC = 2000
GRID_TC = N // ROWS_TC


def _tc1_body(x_ref, lo_ref, hi_ref, deg_ref, ws_ref, wl_ref, wh_ref, b_ref,
              out_lo_ref, out_hi_ref):
    agg = (jnp.dot(lo_ref[...], wl_ref[...], preferred_element_type=jnp.float32)
           + jnp.dot(hi_ref[...], wh_ref[...], preferred_element_type=jnp.float32))
    inv = 1.0 / jnp.maximum(deg_ref[...][:, :1], 1.0)
    h = (jnp.dot(x_ref[...], ws_ref[...], preferred_element_type=jnp.float32)
         + inv * agg + b_ref[...])
    h = jnp.maximum(h, 0.0)
    out_lo_ref[...] = h[:, :DH]
    out_hi_ref[...] = h[:, DH:]


def _tc2_body(xlo_ref, xhi_ref, lo_ref, hi_ref, deg_ref,
              wslo_ref, wshi_ref, wl_ref, wh_ref, b_ref, out_ref):
    agg = (jnp.dot(lo_ref[...], wl_ref[...], preferred_element_type=jnp.float32)
           + jnp.dot(hi_ref[...], wh_ref[...], preferred_element_type=jnp.float32))
    inv = 1.0 / jnp.maximum(deg_ref[...][:, :1], 1.0)
    h = (jnp.dot(xlo_ref[...], wslo_ref[...], preferred_element_type=jnp.float32)
         + jnp.dot(xhi_ref[...], wshi_ref[...], preferred_element_type=jnp.float32)
         + inv * agg + b_ref[...])
    out_ref[...] = h


def _row_spec(cols):
    return pl.BlockSpec((ROWS_TC, cols), lambda i: (i, 0))


def _full_spec(r, c):
    return pl.BlockSpec((r, c), lambda i: (0, 0))


_tc1_call = pl.pallas_call(
    _tc1_body,
    grid=(GRID_TC,),
    in_specs=[_row_spec(D), _row_spec(DH), _row_spec(DH), _row_spec(DH),
              _full_spec(D, D), _full_spec(DH, D), _full_spec(DH, D),
              _full_spec(1, D)],
    out_specs=(_row_spec(DH), _row_spec(DH)),
    out_shape=(jax.ShapeDtypeStruct((N, DH), jnp.float32),
               jax.ShapeDtypeStruct((N, DH), jnp.float32)),
)

_tc2_call = pl.pallas_call(
    _tc2_body,
    grid=(GRID_TC,),
    in_specs=[_row_spec(DH), _row_spec(DH), _row_spec(DH), _row_spec(DH),
              _row_spec(DH), _full_spec(DH, D), _full_spec(DH, D),
              _full_spec(DH, D), _full_spec(DH, D), _full_spec(1, D)],
    out_specs=_row_spec(D),
    out_shape=jax.ShapeDtypeStruct((N, D), jnp.float32),
)


def _edge_blocks(ei):
    src = jnp.concatenate([ei[0].astype(jnp.int32),
                           jnp.zeros((EPAD - E,), jnp.int32)])
    dst = jnp.concatenate([ei[1].astype(jnp.int32),
                           jnp.full((EPAD - E,), N, jnp.int32)])
    s3 = src.reshape(NSUP_TOT, SUBS, CHUNK)
    d3 = dst.reshape(NSUP_TOT, SUBS, CHUNK)
    return jnp.concatenate([s3, d3], axis=1)


def kernel(x_user, x_item, edge_index_rates, edge_index_rated_by,
           W1_rates_self, W1_rates_neigh, W1_rb_self, W1_rb_neigh,
           W2_rates_self, W2_rates_neigh, W2_rb_self, W2_rb_neigh,
           b1_rates, b1_rb, b2_rates, b2_rb):
    e_r = _edge_blocks(edge_index_rates)
    e_b = _edge_blocks(edge_index_rated_by)

    xu_lo, xu_hi = x_user[:, :DH], x_user[:, DH:]
    xi_lo, xi_hi = x_item[:, :DH], x_item[:, DH:]

    deg_r, deg_b = _deg_call(e_r, e_b)

    b1r = b1_rates.reshape(1, D)
    b1b = b1_rb.reshape(1, D)
    b2r = b2_rates.reshape(1, D)
    b2b = b2_rb.reshape(1, D)

    a1i_lo, a1i_hi = _agg_call(xu_lo, xu_hi, e_r)
    a1u_lo, a1u_hi = _agg_call(xi_lo, xi_hi, e_b)
    hi_lo, hi_hi = _tc1_call(x_item, a1i_lo, a1i_hi, deg_r,
                             W1_rates_self, W1_rates_neigh[:DH],
                             W1_rates_neigh[DH:], b1r)
    hu_lo, hu_hi = _tc1_call(x_user, a1u_lo, a1u_hi, deg_b,
                             W1_rb_self, W1_rb_neigh[:DH],
                             W1_rb_neigh[DH:], b1b)

    a2i_lo, a2i_hi = _agg_call(hu_lo, hu_hi, e_r)
    a2u_lo, a2u_hi = _agg_call(hi_lo, hi_hi, e_b)
    h_item2 = _tc2_call(hi_lo, hi_hi, a2i_lo, a2i_hi, deg_r,
                        W2_rates_self[:DH], W2_rates_self[DH:],
                        W2_rates_neigh[:DH], W2_rates_neigh[DH:], b2r)
    h_user2 = _tc2_call(hu_lo, hu_hi, a2u_lo, a2u_hi, deg_b,
                        W2_rb_self[:DH], W2_rb_self[DH:],
                        W2_rb_neigh[:DH], W2_rb_neigh[DH:], b2b)
    return (h_user2, h_item2)


# speedup vs baseline: 1.5085x; 1.3768x over previous
import jax
import jax.numpy as jnp
from jax import lax
from jax.experimental import pallas as pl
from jax.experimental.pallas import tpu as pltpu
from jax.experimental.pallas import tpu_sc as plsc

N = 100000
D = 32
DH = 16
E = 1600000
CHUNK = 128
SUBS = 8
NTILES = 16
NSUP = 98
KITER = NSUP // 2
NSUP_TOT = NTILES * NSUP
EPAD = NSUP_TOT * SUBS * CHUNK
ACC_ROWS = 100096
RPT = ACC_ROWS // NTILES
ZROWS = 782

_mesh = plsc.VectorSubcoreMesh(core_axis_name="c", subcore_axis_name="s")
_sc_params = pltpu.CompilerParams(use_tc_tiling_on_sc=False)


def _zero_acc(zbuf, acc, t):
    def fill_zero(i, carry):
        zbuf[i, :] = jnp.zeros((DH,), jnp.float32)
        return carry

    lax.fori_loop(0, ZROWS, fill_zero, 0)

    def zero_copy(k, carry):
        pltpu.sync_copy(zbuf, acc.at[pl.ds(t * RPT + k * ZROWS, ZROWS)])
        return carry

    lax.fori_loop(0, RPT // ZROWS, zero_copy, 0)


def _writeout(c, t, acc, out_lo, out_hi):
    @pl.when(c == 0)
    def _():
        pltpu.sync_copy(acc.at[pl.ds(t * RPT, RPT)],
                        out_lo.at[pl.ds(t * RPT, RPT)])

    @pl.when(c == 1)
    def _():
        pltpu.sync_copy(acc.at[pl.ds(t * RPT, RPT)],
                        out_hi.at[pl.ds(t * RPT, RPT)])


def _deg_phase(c, t, eidx_r, eidx_b, deg_r, deg_b,
               eA, eB, ones_v, zbuf, acc, isemA, isemB, ssemA, ssemB):
    def fill_ones(i, carry):
        ones_v[i, :] = jnp.ones((DH,), jnp.float32)
        return carry

    lax.fori_loop(0, CHUNK, fill_ones, 0)
    _zero_acc(zbuf, acc, t)
    plsc.subcore_barrier()

    def run(eidx_hbm):
        base = t * NSUP

        def scatters(e, sem):
            for j in range(SUBS):
                pltpu.async_copy(ones_v, acc.at[e.at[SUBS + j]], sem, add=True)

        def wait_scatters(e, sem):
            for j in range(SUBS):
                pltpu.make_async_copy(ones_v, acc.at[e.at[SUBS + j]],
                                      sem).wait()

        pltpu.sync_copy(eidx_hbm.at[base], eA)

        def body(k, carry):
            gB = 2 * k + 1

            @pl.when(k > 0)
            def _():
                wait_scatters(eB, ssemB)

            idx_b = pltpu.async_copy(eidx_hbm.at[base + gB], eB, isemB)
            scatters(eA, ssemA)
            idx_b.wait()
            wait_scatters(eA, ssemA)

            @pl.when(k < KITER - 1)
            def _():
                pltpu.async_copy(eidx_hbm.at[base + gB + 1], eA, isemA)

            scatters(eB, ssemB)

            @pl.when(k < KITER - 1)
            def _():
                pltpu.make_async_copy(eidx_hbm.at[base + gB + 1], eA,
                                      isemA).wait()

            return carry

        lax.fori_loop(0, KITER, body, 0)
        wait_scatters(eB, ssemB)

    @pl.when(c == 0)
    def _():
        run(eidx_r)

    @pl.when(c == 1)
    def _():
        run(eidx_b)

    plsc.subcore_barrier()
    _writeout(c, t, acc, deg_r, deg_b)


def _agg_phase(c, t, tlo, thi, eidx_hbm, out_lo, out_hi,
               eA, eB, rows, zbuf, acc, isemA, isemB, gsem, ssem):
    _zero_acc(zbuf, acc, t)
    plsc.subcore_barrier()

    def run(table):
        base = t * NSUP

        def srow(e, m):
            return e.at[SUBS + m]

        def g_issue(e, m, p):
            pltpu.async_copy(table.at[e.at[m]], rows[p], gsem[p])

        def g_wait(e, m, p):
            pltpu.make_async_copy(table.at[e.at[m]], rows[p], gsem[p]).wait()

        def s_issue(e, m, p):
            pltpu.async_copy(rows[p], acc.at[srow(e, m)], ssem[p], add=True)

        def s_wait(e, m, p):
            pltpu.make_async_copy(rows[p], acc.at[srow(e, m)], ssem[p]).wait()

        pltpu.sync_copy(eidx_hbm.at[base], eA)
        g_issue(eA, 0, 0)
        g_issue(eA, 1, 1)

        def body(k, carry):
            for j in range(16):
                p = j % 4
                q = (j + 2) % 4
                if j < 2:
                    @pl.when(k > 0)
                    def _(j=j, q=q):
                        s_wait(eB, j + 6, q)
                else:
                    jm = j - 2
                    s_wait(eA if jm < 8 else eB, jm % 8, q)
                if j == 2:
                    pltpu.async_copy(eidx_hbm.at[base + 2 * k + 1], eB, isemB)
                if j == 6:
                    pltpu.make_async_copy(eidx_hbm.at[base + 2 * k + 1], eB,
                                          isemB).wait()
                jp = j + 2
                if jp < 8:
                    g_issue(eA, jp, q)
                elif jp < 16:
                    g_issue(eB, jp - 8, q)
                else:
                    if j == 14:
                        @pl.when(k < KITER - 1)
                        def _():
                            pltpu.make_async_copy(
                                eidx_hbm.at[base + 2 * k + 2], eA,
                                isemA).wait()
                            g_issue(eA, 0, 0)
                    else:
                        @pl.when(k < KITER - 1)
                        def _():
                            g_issue(eA, 1, 1)
                if j < 8:
                    g_wait(eA, j, p)
                    s_issue(eA, j, p)
                else:
                    g_wait(eB, j - 8, p)
                    s_issue(eB, j - 8, p)
                if j == 9:
                    @pl.when(k < KITER - 1)
                    def _():
                        pltpu.async_copy(eidx_hbm.at[base + 2 * k + 2], eA,
                                         isemA)
            return carry

        lax.fori_loop(0, KITER, body, 0)
        s_wait(eB, 6, 2)
        s_wait(eB, 7, 3)

    @pl.when(c == 0)
    def _():
        run(tlo)

    @pl.when(c == 1)
    def _():
        run(thi)

    plsc.subcore_barrier()
    _writeout(c, t, acc, out_lo, out_hi)


_SC_SCRATCH = [
    pltpu.VMEM((2 * SUBS, CHUNK), jnp.int32),
    pltpu.VMEM((2 * SUBS, CHUNK), jnp.int32),
    pltpu.VMEM((CHUNK, DH), jnp.float32),
    pltpu.VMEM((CHUNK, DH), jnp.float32),
    pltpu.VMEM((CHUNK, DH), jnp.float32),
    pltpu.VMEM((CHUNK, DH), jnp.float32),
    pltpu.VMEM((CHUNK, DH), jnp.float32),
    pltpu.VMEM((ZROWS, DH), jnp.float32),
    pltpu.VMEM_SHARED((ACC_ROWS, DH), jnp.float32),
] + [pltpu.SemaphoreType.DMA] * 10


def _sc1_body(xu_lo, xu_hi, xi_lo, xi_hi, e_r, e_b,
              deg_r, deg_b, a1i_lo, a1i_hi, a1u_lo, a1u_hi,
              eA, eB, r0, r1, r2, r3, ones_v, zbuf, acc,
              isemA, isemB, gs0, gs1, gs2, gs3, ss0, ss1, ss2, ss3):
    c = lax.axis_index("c")
    t = lax.axis_index("s")
    rows = (r0, r1, r2, r3)
    gsem = (gs0, gs1, gs2, gs3)
    ssem = (ss0, ss1, ss2, ss3)
    _deg_phase(c, t, e_r, e_b, deg_r, deg_b,
               eA, eB, ones_v, zbuf, acc, isemA, isemB, ss0, ss1)
    _agg_phase(c, t, xu_lo, xu_hi, e_r, a1i_lo, a1i_hi,
               eA, eB, rows, zbuf, acc, isemA, isemB, gsem, ssem)
    _agg_phase(c, t, xi_lo, xi_hi, e_b, a1u_lo, a1u_hi,
               eA, eB, rows, zbuf, acc, isemA, isemB, gsem, ssem)


def _sc2_body(hu_lo, hu_hi, hi_lo, hi_hi, e_r, e_b,
              a2i_lo, a2i_hi, a2u_lo, a2u_hi,
              eA, eB, r0, r1, r2, r3, ones_v, zbuf, acc,
              isemA, isemB, gs0, gs1, gs2, gs3, ss0, ss1, ss2, ss3):
    c = lax.axis_index("c")
    t = lax.axis_index("s")
    rows = (r0, r1, r2, r3)
    gsem = (gs0, gs1, gs2, gs3)
    ssem = (ss0, ss1, ss2, ss3)
    _agg_phase(c, t, hu_lo, hu_hi, e_r, a2i_lo, a2i_hi,
               eA, eB, rows, zbuf, acc, isemA, isemB, gsem, ssem)
    _agg_phase(c, t, hi_lo, hi_hi, e_b, a2u_lo, a2u_hi,
               eA, eB, rows, zbuf, acc, isemA, isemB, gsem, ssem)


_half = jax.ShapeDtypeStruct((ACC_ROWS, DH), jnp.float32)

_sc1_call = pl.kernel(
    _sc1_body,
    out_type=(_half,) * 6,
    mesh=_mesh,
    scratch_types=list(_SC_SCRATCH),
    compiler_params=_sc_params,
)

_sc2_call = pl.kernel(
    _sc2_body,
    out_type=(_half,) * 4,
    mesh=_mesh,
    scratch_types=list(_SC_SCRATCH),
    compiler_params=_sc_params,
)

ROWS_TC = 2000
GRID_TC = N // ROWS_TC


def _tc1_body(x_ref, lo_ref, hi_ref, deg_ref, ws_ref, wl_ref, wh_ref, b_ref,
              out_lo_ref, out_hi_ref):
    agg = (jnp.dot(lo_ref[...], wl_ref[...], preferred_element_type=jnp.float32)
           + jnp.dot(hi_ref[...], wh_ref[...], preferred_element_type=jnp.float32))
    inv = 1.0 / jnp.maximum(deg_ref[...][:, :1], 1.0)
    h = (jnp.dot(x_ref[...], ws_ref[...], preferred_element_type=jnp.float32)
         + inv * agg + b_ref[...])
    h = jnp.maximum(h, 0.0)
    out_lo_ref[...] = h[:, :DH]
    out_hi_ref[...] = h[:, DH:]


def _tc2_body(xlo_ref, xhi_ref, lo_ref, hi_ref, deg_ref,
              wslo_ref, wshi_ref, wl_ref, wh_ref, b_ref, out_ref):
    agg = (jnp.dot(lo_ref[...], wl_ref[...], preferred_element_type=jnp.float32)
           + jnp.dot(hi_ref[...], wh_ref[...], preferred_element_type=jnp.float32))
    inv = 1.0 / jnp.maximum(deg_ref[...][:, :1], 1.0)
    h = (jnp.dot(xlo_ref[...], wslo_ref[...], preferred_element_type=jnp.float32)
         + jnp.dot(xhi_ref[...], wshi_ref[...], preferred_element_type=jnp.float32)
         + inv * agg + b_ref[...])
    out_ref[...] = h


def _row_spec(cols):
    return pl.BlockSpec((ROWS_TC, cols), lambda i: (i, 0))


def _full_spec(r, c):
    return pl.BlockSpec((r, c), lambda i: (0, 0))


_tc1_call = pl.pallas_call(
    _tc1_body,
    grid=(GRID_TC,),
    in_specs=[_row_spec(D), _row_spec(DH), _row_spec(DH), _row_spec(DH),
              _full_spec(D, D), _full_spec(DH, D), _full_spec(DH, D),
              _full_spec(1, D)],
    out_specs=(_row_spec(DH), _row_spec(DH)),
    out_shape=(jax.ShapeDtypeStruct((N, DH), jnp.float32),
               jax.ShapeDtypeStruct((N, DH), jnp.float32)),
)

_tc2_call = pl.pallas_call(
    _tc2_body,
    grid=(GRID_TC,),
    in_specs=[_row_spec(DH), _row_spec(DH), _row_spec(DH), _row_spec(DH),
              _row_spec(DH), _full_spec(DH, D), _full_spec(DH, D),
              _full_spec(DH, D), _full_spec(DH, D), _full_spec(1, D)],
    out_specs=_row_spec(D),
    out_shape=jax.ShapeDtypeStruct((N, D), jnp.float32),
)


def _edge_blocks(ei):
    src = jnp.concatenate([ei[0].astype(jnp.int32),
                           jnp.zeros((EPAD - E,), jnp.int32)])
    dst = jnp.concatenate([ei[1].astype(jnp.int32),
                           jnp.full((EPAD - E,), N, jnp.int32)])
    s3 = src.reshape(NSUP_TOT, SUBS, CHUNK)
    d3 = dst.reshape(NSUP_TOT, SUBS, CHUNK)
    return jnp.concatenate([s3, d3], axis=1)


def kernel(x_user, x_item, edge_index_rates, edge_index_rated_by,
           W1_rates_self, W1_rates_neigh, W1_rb_self, W1_rb_neigh,
           W2_rates_self, W2_rates_neigh, W2_rb_self, W2_rb_neigh,
           b1_rates, b1_rb, b2_rates, b2_rb):
    e_r = _edge_blocks(edge_index_rates)
    e_b = _edge_blocks(edge_index_rated_by)

    xu_lo, xu_hi = x_user[:, :DH], x_user[:, DH:]
    xi_lo, xi_hi = x_item[:, :DH], x_item[:, DH:]

    b1r = b1_rates.reshape(1, D)
    b1b = b1_rb.reshape(1, D)
    b2r = b2_rates.reshape(1, D)
    b2b = b2_rb.reshape(1, D)

    deg_r, deg_b, a1i_lo, a1i_hi, a1u_lo, a1u_hi = _sc1_call(
        xu_lo, xu_hi, xi_lo, xi_hi, e_r, e_b)

    hi_lo, hi_hi = _tc1_call(x_item, a1i_lo, a1i_hi, deg_r,
                             W1_rates_self, W1_rates_neigh[:DH],
                             W1_rates_neigh[DH:], b1r)
    hu_lo, hu_hi = _tc1_call(x_user, a1u_lo, a1u_hi, deg_b,
                             W1_rb_self, W1_rb_neigh[:DH],
                             W1_rb_neigh[DH:], b1b)

    a2i_lo, a2i_hi, a2u_lo, a2u_hi = _sc2_call(
        hu_lo, hu_hi, hi_lo, hi_hi, e_r, e_b)

    h_item2 = _tc2_call(hi_lo, hi_hi, a2i_lo, a2i_hi, deg_r,
                        W2_rates_self[:DH], W2_rates_self[DH:],
                        W2_rates_neigh[:DH], W2_rates_neigh[DH:], b2r)
    h_user2 = _tc2_call(hu_lo, hu_hi, a2u_lo, a2u_hi, deg_b,
                        W2_rb_self[:DH], W2_rb_self[DH:],
                        W2_rb_neigh[:DH], W2_rb_neigh[DH:], b2b)
    return (h_user2, h_item2)


# speedup vs baseline: 1.9503x; 1.2929x over previous
import jax
import jax.numpy as jnp
from jax import lax
from jax.experimental import pallas as pl
from jax.experimental.pallas import tpu as pltpu
from jax.experimental.pallas import tpu_sc as plsc

N = 100000
D = 32
DH = 16
E = 1600000
CHUNK = 128
SUBS = 8
NTILES = 16
NSUP = 98
KITER = NSUP // 2
NSUP_TOT = NTILES * NSUP
EPAD = NSUP_TOT * SUBS * CHUNK
ACC_ROWS = 100096
RPT = ACC_ROWS // NTILES
ZROWS = 782

_mesh = plsc.VectorSubcoreMesh(core_axis_name="c", subcore_axis_name="s")
_sc_params = pltpu.CompilerParams(use_tc_tiling_on_sc=False)


def _zero_acc(zbuf, acc, t):
    def fill_zero(i, carry):
        zbuf[i, :] = jnp.zeros((DH,), jnp.float32)
        return carry

    lax.fori_loop(0, ZROWS, fill_zero, 0)

    def zero_copy(k, carry):
        pltpu.sync_copy(zbuf, acc.at[pl.ds(t * RPT + k * ZROWS, ZROWS)])
        return carry

    lax.fori_loop(0, RPT // ZROWS, zero_copy, 0)


def _writeout(c, t, acc, out_lo, out_hi):
    @pl.when(c == 0)
    def _():
        pltpu.sync_copy(acc.at[pl.ds(t * RPT, RPT)],
                        out_lo.at[pl.ds(t * RPT, RPT)])

    @pl.when(c == 1)
    def _():
        pltpu.sync_copy(acc.at[pl.ds(t * RPT, RPT)],
                        out_hi.at[pl.ds(t * RPT, RPT)])


def _deg_phase(c, t, eidx_r, eidx_b, deg_r, deg_b,
               eA, eB, ones_v, zbuf, acc, isemA, isemB, ssemA, ssemB):
    def fill_ones(i, carry):
        ones_v[i, :] = jnp.ones((DH,), jnp.float32)
        return carry

    lax.fori_loop(0, CHUNK, fill_ones, 0)
    _zero_acc(zbuf, acc, t)
    plsc.subcore_barrier()

    def run(eidx_hbm):
        base = t * NSUP

        def scatters(e, sem):
            for j in range(SUBS):
                pltpu.async_copy(ones_v, acc.at[e.at[SUBS + j]], sem, add=True)

        def wait_scatters(e, sem):
            for j in range(SUBS):
                pltpu.make_async_copy(ones_v, acc.at[e.at[SUBS + j]],
                                      sem).wait()

        pltpu.sync_copy(eidx_hbm.at[base], eA)

        def body(k, carry):
            gB = 2 * k + 1

            @pl.when(k > 0)
            def _():
                wait_scatters(eB, ssemB)

            idx_b = pltpu.async_copy(eidx_hbm.at[base + gB], eB, isemB)
            scatters(eA, ssemA)
            idx_b.wait()
            wait_scatters(eA, ssemA)

            @pl.when(k < KITER - 1)
            def _():
                pltpu.async_copy(eidx_hbm.at[base + gB + 1], eA, isemA)

            scatters(eB, ssemB)

            @pl.when(k < KITER - 1)
            def _():
                pltpu.make_async_copy(eidx_hbm.at[base + gB + 1], eA,
                                      isemA).wait()

            return carry

        lax.fori_loop(0, KITER, body, 0)
        wait_scatters(eB, ssemB)

    @pl.when(c == 0)
    def _():
        run(eidx_r)

    @pl.when(c == 1)
    def _():
        run(eidx_b)

    plsc.subcore_barrier()
    _writeout(c, t, acc, deg_r, deg_b)


def _agg_phase(c, t, tlo, thi, eidx_hbm, out_lo, out_hi,
               eA, eB, rows, zbuf, acc, isemA, isemB, gsem, ssem):
    _zero_acc(zbuf, acc, t)
    plsc.subcore_barrier()

    def run(table):
        base = t * NSUP

        def srow(e, m):
            return e.at[SUBS + m]

        def g_issue(e, m, p):
            pltpu.async_copy(table.at[e.at[m]], rows[p], gsem[p])

        def g_wait(e, m, p):
            pltpu.make_async_copy(table.at[e.at[m]], rows[p], gsem[p]).wait()

        def s_issue(e, m, p):
            pltpu.async_copy(rows[p], acc.at[srow(e, m)], ssem[p], add=True)

        def s_wait(e, m, p):
            pltpu.make_async_copy(rows[p], acc.at[srow(e, m)], ssem[p]).wait()

        pltpu.sync_copy(eidx_hbm.at[base], eA)
        g_issue(eA, 0, 0)
        g_issue(eA, 1, 1)

        def body(k, carry):
            for j in range(16):
                p = j % 4
                q = (j + 2) % 4
                if j < 2:
                    @pl.when(k > 0)
                    def _(j=j, q=q):
                        s_wait(eB, j + 6, q)
                else:
                    jm = j - 2
                    s_wait(eA if jm < 8 else eB, jm % 8, q)
                if j == 2:
                    pltpu.async_copy(eidx_hbm.at[base + 2 * k + 1], eB, isemB)
                if j == 6:
                    pltpu.make_async_copy(eidx_hbm.at[base + 2 * k + 1], eB,
                                          isemB).wait()
                jp = j + 2
                if jp < 8:
                    g_issue(eA, jp, q)
                elif jp < 16:
                    g_issue(eB, jp - 8, q)
                else:
                    if j == 14:
                        @pl.when(k < KITER - 1)
                        def _():
                            pltpu.make_async_copy(
                                eidx_hbm.at[base + 2 * k + 2], eA,
                                isemA).wait()
                            g_issue(eA, 0, 0)
                    else:
                        @pl.when(k < KITER - 1)
                        def _():
                            g_issue(eA, 1, 1)
                if j < 8:
                    g_wait(eA, j, p)
                    s_issue(eA, j, p)
                else:
                    g_wait(eB, j - 8, p)
                    s_issue(eB, j - 8, p)
                if j == 9:
                    @pl.when(k < KITER - 1)
                    def _():
                        pltpu.async_copy(eidx_hbm.at[base + 2 * k + 2], eA,
                                         isemA)
            return carry

        lax.fori_loop(0, KITER, body, 0)
        s_wait(eB, 6, 2)
        s_wait(eB, 7, 3)

    @pl.when(c == 0)
    def _():
        run(tlo)

    @pl.when(c == 1)
    def _():
        run(thi)

    plsc.subcore_barrier()
    _writeout(c, t, acc, out_lo, out_hi)


_SC_SCRATCH = [
    pltpu.VMEM((2 * SUBS, CHUNK), jnp.int32),
    pltpu.VMEM((2 * SUBS, CHUNK), jnp.int32),
    pltpu.VMEM((CHUNK, DH), jnp.float32),
    pltpu.VMEM((CHUNK, DH), jnp.float32),
    pltpu.VMEM((CHUNK, DH), jnp.float32),
    pltpu.VMEM((CHUNK, DH), jnp.float32),
    pltpu.VMEM((CHUNK, DH), jnp.float32),
    pltpu.VMEM((ZROWS, DH), jnp.float32),
    pltpu.VMEM_SHARED((ACC_ROWS, DH), jnp.float32),
] + [pltpu.SemaphoreType.DMA] * 10


def _deg_body(e_r, e_b, deg_r, deg_b,
              eA, eB, r0, r1, r2, r3, ones_v, zbuf, acc,
              isemA, isemB, gs0, gs1, gs2, gs3, ss0, ss1, ss2, ss3):
    c = lax.axis_index("c")
    t = lax.axis_index("s")
    _deg_phase(c, t, e_r, e_b, deg_r, deg_b,
               eA, eB, ones_v, zbuf, acc, isemA, isemB, ss0, ss1)


def _agg_body(tlo, thi, eidx, out_lo, out_hi,
              eA, eB, r0, r1, r2, r3, ones_v, zbuf, acc,
              isemA, isemB, gs0, gs1, gs2, gs3, ss0, ss1, ss2, ss3):
    c = lax.axis_index("c")
    t = lax.axis_index("s")
    rows = (r0, r1, r2, r3)
    gsem = (gs0, gs1, gs2, gs3)
    ssem = (ss0, ss1, ss2, ss3)
    _agg_phase(c, t, tlo, thi, eidx, out_lo, out_hi,
               eA, eB, rows, zbuf, acc, isemA, isemB, gsem, ssem)


_half = jax.ShapeDtypeStruct((ACC_ROWS, DH), jnp.float32)

_deg_call = pl.kernel(
    _deg_body,
    out_type=(_half,) * 2,
    mesh=_mesh,
    scratch_types=list(_SC_SCRATCH),
    compiler_params=_sc_params,
)

_agg_call = pl.kernel(
    _agg_body,
    out_type=(_half,) * 2,
    mesh=_mesh,
    scratch_types=list(_SC_SCRATCH),
    compiler_params=_sc_params,
)

ROWS_TC = 2000
GRID_TC = N // ROWS_TC


def _tc1_body(x_ref, lo_ref, hi_ref, deg_ref, ws_ref, wl_ref, wh_ref, b_ref,
              out_lo_ref, out_hi_ref):
    agg = (jnp.dot(lo_ref[...], wl_ref[...], preferred_element_type=jnp.float32)
           + jnp.dot(hi_ref[...], wh_ref[...], preferred_element_type=jnp.float32))
    inv = 1.0 / jnp.maximum(deg_ref[...][:, :1], 1.0)
    h = (jnp.dot(x_ref[...], ws_ref[...], preferred_element_type=jnp.float32)
         + inv * agg + b_ref[...])
    h = jnp.maximum(h, 0.0)
    out_lo_ref[...] = h[:, :DH]
    out_hi_ref[...] = h[:, DH:]


def _tc2_body(xlo_ref, xhi_ref, lo_ref, hi_ref, deg_ref,
              wslo_ref, wshi_ref, wl_ref, wh_ref, b_ref, out_ref):
    agg = (jnp.dot(lo_ref[...], wl_ref[...], preferred_element_type=jnp.float32)
           + jnp.dot(hi_ref[...], wh_ref[...], preferred_element_type=jnp.float32))
    inv = 1.0 / jnp.maximum(deg_ref[...][:, :1], 1.0)
    h = (jnp.dot(xlo_ref[...], wslo_ref[...], preferred_element_type=jnp.float32)
         + jnp.dot(xhi_ref[...], wshi_ref[...], preferred_element_type=jnp.float32)
         + inv * agg + b_ref[...])
    out_ref[...] = h


def _row_spec(cols):
    return pl.BlockSpec((ROWS_TC, cols), lambda i: (i, 0))


def _full_spec(r, c):
    return pl.BlockSpec((r, c), lambda i: (0, 0))


_tc1_call = pl.pallas_call(
    _tc1_body,
    grid=(GRID_TC,),
    in_specs=[_row_spec(D), _row_spec(DH), _row_spec(DH), _row_spec(DH),
              _full_spec(D, D), _full_spec(DH, D), _full_spec(DH, D),
              _full_spec(1, D)],
    out_specs=(_row_spec(DH), _row_spec(DH)),
    out_shape=(jax.ShapeDtypeStruct((N, DH), jnp.float32),
               jax.ShapeDtypeStruct((N, DH), jnp.float32)),
)

_tc2_call = pl.pallas_call(
    _tc2_body,
    grid=(GRID_TC,),
    in_specs=[_row_spec(DH), _row_spec(DH), _row_spec(DH), _row_spec(DH),
              _row_spec(DH), _full_spec(DH, D), _full_spec(DH, D),
              _full_spec(DH, D), _full_spec(DH, D), _full_spec(1, D)],
    out_specs=_row_spec(D),
    out_shape=jax.ShapeDtypeStruct((N, D), jnp.float32),
)


def _edge_blocks(ei):
    src = jnp.concatenate([ei[0].astype(jnp.int32),
                           jnp.zeros((EPAD - E,), jnp.int32)])
    dst = jnp.concatenate([ei[1].astype(jnp.int32),
                           jnp.full((EPAD - E,), N, jnp.int32)])
    s3 = src.reshape(NSUP_TOT, SUBS, CHUNK)
    d3 = dst.reshape(NSUP_TOT, SUBS, CHUNK)
    return jnp.concatenate([s3, d3], axis=1)


def kernel(x_user, x_item, edge_index_rates, edge_index_rated_by,
           W1_rates_self, W1_rates_neigh, W1_rb_self, W1_rb_neigh,
           W2_rates_self, W2_rates_neigh, W2_rb_self, W2_rb_neigh,
           b1_rates, b1_rb, b2_rates, b2_rb):
    e_r = _edge_blocks(edge_index_rates)
    e_b = _edge_blocks(edge_index_rated_by)

    xu_lo, xu_hi = x_user[:, :DH], x_user[:, DH:]
    xi_lo, xi_hi = x_item[:, :DH], x_item[:, DH:]

    b1r = b1_rates.reshape(1, D)
    b1b = b1_rb.reshape(1, D)
    b2r = b2_rates.reshape(1, D)
    b2b = b2_rb.reshape(1, D)

    deg_r, deg_b = _deg_call(e_r, e_b)
    a1i_lo, a1i_hi = _agg_call(xu_lo, xu_hi, e_r)
    a1u_lo, a1u_hi = _agg_call(xi_lo, xi_hi, e_b)

    hi_lo, hi_hi = _tc1_call(x_item, a1i_lo, a1i_hi, deg_r,
                             W1_rates_self, W1_rates_neigh[:DH],
                             W1_rates_neigh[DH:], b1r)
    hu_lo, hu_hi = _tc1_call(x_user, a1u_lo, a1u_hi, deg_b,
                             W1_rb_self, W1_rb_neigh[:DH],
                             W1_rb_neigh[DH:], b1b)

    a2i_lo, a2i_hi = _agg_call(hu_lo, hu_hi, e_r)
    a2u_lo, a2u_hi = _agg_call(hi_lo, hi_hi, e_b)

    h_item2 = _tc2_call(hi_lo, hi_hi, a2i_lo, a2i_hi, deg_r,
                        W2_rates_self[:DH], W2_rates_self[DH:],
                        W2_rates_neigh[:DH], W2_rates_neigh[DH:], b2r)
    h_user2 = _tc2_call(hu_lo, hu_hi, a2u_lo, a2u_hi, deg_b,
                        W2_rb_self[:DH], W2_rb_self[DH:],
                        W2_rb_neigh[:DH], W2_rb_neigh[DH:], b2b)
    return (h_user2, h_item2)
